# baseline (speedup 1.0000x reference)
import functools

import jax
import jax.numpy as jnp
from jax import lax
from jax.experimental import pallas as pl
from jax.experimental.pallas import tpu as pltpu
from jax.experimental.pallas import tpu_sc as plsc

EMBED_DIM = 64
NUM_CORES = 2
NUM_SUBCORES = 16
NUM_WORKERS = NUM_CORES * NUM_SUBCORES
CHUNK = 128


def _gather_call(b_total: int, n_chunks: int):
    mesh = plsc.VectorSubcoreMesh(core_axis_name="c", subcore_axis_name="s")

    @functools.partial(
        pl.kernel,
        out_type=jax.ShapeDtypeStruct((b_total, EMBED_DIM), jnp.float32),
        mesh=mesh,
        scratch_types=[
            pltpu.VMEM((CHUNK,), jnp.int32),
            pltpu.VMEM((CHUNK, EMBED_DIM), jnp.float32),
            pltpu.SemaphoreType.DMA,
        ],
        compiler_params=pltpu.CompilerParams(use_tc_tiling_on_sc=False),
    )
    def k(idx_hbm, table_hbm, out_hbm, idx_v, rows_v, gsem):
        wid = lax.axis_index("s") * NUM_CORES + lax.axis_index("c")
        base = wid * (n_chunks * CHUNK)

        def body(j, carry):
            row0 = base + j * CHUNK
            pltpu.sync_copy(idx_hbm.at[pl.ds(row0, CHUNK)], idx_v)
            pltpu.async_copy(table_hbm.at[idx_v], rows_v, gsem).wait()
            pltpu.sync_copy(rows_v, out_hbm.at[pl.ds(row0, CHUNK)])
            return carry

        lax.fori_loop(0, n_chunks, body, 0)

    return k


def kernel(x, embedding):
    batch, hist = x.shape
    b_total = batch * hist
    per_worker = b_total // NUM_WORKERS
    n_chunks = per_worker // CHUNK
    idx = x.reshape(b_total).astype(jnp.int32)
    out = _gather_call(b_total, n_chunks)(idx, embedding)
    return out.reshape(batch, hist, EMBED_DIM)


# speedup vs baseline: 1.1474x; 1.1474x over previous
import functools

import jax
import jax.numpy as jnp
from jax import lax
from jax.experimental import pallas as pl
from jax.experimental.pallas import tpu as pltpu
from jax.experimental.pallas import tpu_sc as plsc

EMBED_DIM = 64
NUM_CORES = 2
NUM_SUBCORES = 16
NUM_WORKERS = NUM_CORES * NUM_SUBCORES
CHUNK = 128
NBUF = 4


def _gather_call(b_total: int, n_chunks: int):
    mesh = plsc.VectorSubcoreMesh(core_axis_name="c", subcore_axis_name="s")
    n_grp = n_chunks // NBUF

    @functools.partial(
        pl.kernel,
        out_type=jax.ShapeDtypeStruct((b_total, EMBED_DIM), jnp.float32),
        mesh=mesh,
        scratch_types=[
            pltpu.VMEM((NBUF, CHUNK), jnp.int32),
            pltpu.VMEM((NBUF, CHUNK, EMBED_DIM), jnp.float32),
            pltpu.SemaphoreType.DMA((NBUF,)),
            pltpu.SemaphoreType.DMA((NBUF,)),
        ],
        compiler_params=pltpu.CompilerParams(use_tc_tiling_on_sc=False),
    )
    def k(idx_hbm, table_hbm, out_hbm, idx_v, rows_v, gsem, osem):
        wid = lax.axis_index("s") * NUM_CORES + lax.axis_index("c")
        base = wid * (n_chunks * CHUNK)

        def gather_chunk(row0, b):
            pltpu.sync_copy(idx_hbm.at[pl.ds(row0, CHUNK)], idx_v.at[b])
            pltpu.async_copy(table_hbm.at[idx_v.at[b]], rows_v.at[b], gsem.at[b])

        def gather_wait(b):
            pltpu.make_async_copy(
                table_hbm.at[idx_v.at[b]], rows_v.at[b], gsem.at[b]
            ).wait()

        def out_start(row0, b):
            pltpu.async_copy(
                rows_v.at[b], out_hbm.at[pl.ds(row0, CHUNK)], osem.at[b]
            )

        def out_wait(row0, b):
            pltpu.make_async_copy(
                rows_v.at[b], out_hbm.at[pl.ds(row0, CHUNK)], osem.at[b]
            ).wait()

        for b in range(NBUF):
            gather_chunk(base + b * CHUNK, b)

        def group(g, carry):
            for b in range(NBUF):
                row0 = base + (g * NBUF + b) * CHUNK
                gather_wait(b)
                out_start(row0, b)

                @pl.when(g < n_grp - 1)
                def _():
                    out_wait(row0, b)
                    gather_chunk(row0 + NBUF * CHUNK, b)

            return carry

        lax.fori_loop(0, n_grp, group, 0)

        for b in range(NBUF):
            out_wait(base + ((n_grp - 1) * NBUF + b) * CHUNK, b)

    return k


def kernel(x, embedding):
    batch, hist = x.shape
    b_total = batch * hist
    per_worker = b_total // NUM_WORKERS
    n_chunks = per_worker // CHUNK
    idx = x.reshape(b_total).astype(jnp.int32)
    out = _gather_call(b_total, n_chunks)(idx, embedding)
    return out.reshape(batch, hist, EMBED_DIM)


# speedup vs baseline: 1.1950x; 1.0415x over previous
import functools

import jax
import jax.numpy as jnp
from jax import lax
from jax.experimental import pallas as pl
from jax.experimental.pallas import tpu as pltpu
from jax.experimental.pallas import tpu_sc as plsc

EMBED_DIM = 64
NUM_CORES = 2
NUM_SUBCORES = 16
NUM_WORKERS = NUM_CORES * NUM_SUBCORES
CHUNK = 128
NBUF = 4


def _gather_call(b_total: int, n_chunks: int):
    mesh = plsc.VectorSubcoreMesh(core_axis_name="c", subcore_axis_name="s")
    n_grp = n_chunks // NBUF

    @functools.partial(
        pl.kernel,
        out_type=jax.ShapeDtypeStruct((b_total, EMBED_DIM), jnp.float32),
        mesh=mesh,
        scratch_types=[
            pltpu.VMEM((n_chunks * CHUNK,), jnp.int32),
            pltpu.VMEM((NBUF, CHUNK, EMBED_DIM), jnp.float32),
            pltpu.SemaphoreType.DMA((NBUF,)),
            pltpu.SemaphoreType.DMA((NBUF,)),
        ],
        compiler_params=pltpu.CompilerParams(use_tc_tiling_on_sc=False),
    )
    def k(idx_hbm, table_hbm, out_hbm, idx_v, rows_v, gsem, osem):
        wid = lax.axis_index("s") * NUM_CORES + lax.axis_index("c")
        base = wid * (n_chunks * CHUNK)

        pltpu.sync_copy(idx_hbm.at[pl.ds(base, n_chunks * CHUNK)], idx_v)

        def gather_chunk(j, b):
            pltpu.async_copy(
                table_hbm.at[idx_v.at[pl.ds(j * CHUNK, CHUNK)]],
                rows_v.at[b],
                gsem.at[b],
            )

        def gather_wait(j, b):
            pltpu.make_async_copy(
                table_hbm.at[idx_v.at[pl.ds(j * CHUNK, CHUNK)]],
                rows_v.at[b],
                gsem.at[b],
            ).wait()

        def out_start(row0, b):
            pltpu.async_copy(
                rows_v.at[b], out_hbm.at[pl.ds(row0, CHUNK)], osem.at[b]
            )

        def out_wait(row0, b):
            pltpu.make_async_copy(
                rows_v.at[b], out_hbm.at[pl.ds(row0, CHUNK)], osem.at[b]
            ).wait()

        for b in range(NBUF):
            gather_chunk(b, b)

        def group(g, carry):
            for b in range(NBUF):
                j = g * NBUF + b
                row0 = base + j * CHUNK
                gather_wait(j, b)
                out_start(row0, b)

                @pl.when(g < n_grp - 1)
                def _():
                    out_wait(row0, b)
                    gather_chunk(j + NBUF, b)

            return carry

        lax.fori_loop(0, n_grp, group, 0)

        for b in range(NBUF):
            out_wait(base + ((n_grp - 1) * NBUF + b) * CHUNK, b)

    return k


def kernel(x, embedding):
    batch, hist = x.shape
    b_total = batch * hist
    per_worker = b_total // NUM_WORKERS
    n_chunks = per_worker // CHUNK
    idx = x.reshape(b_total).astype(jnp.int32)
    out = _gather_call(b_total, n_chunks)(idx, embedding)
    return out.reshape(batch, hist, EMBED_DIM)
